# baseline (speedup 1.0000x reference)
import functools

import jax
import jax.numpy as jnp
import numpy as np
from jax import lax
from jax.experimental import pallas as pl
from jax.experimental.pallas import tpu as pltpu

_F32 = jnp.float32
_BF16 = jnp.bfloat16
_F8 = jnp.float8_e4m3fn


def _dot(a, b, trans_b=False):
    if a.dtype != _F8:
        a = a.astype(_BF16)
    if b.dtype != _F8:
        b = b.astype(_BF16)
    dims = (((1,), (1 if trans_b else 0,)), ((), ()))
    return lax.dot_general(a, b, dims, preferred_element_type=_F32)


def _mm_body(a_ref, b_ref, o_ref, *, act, scale):
    o = _dot(a_ref[...], b_ref[...])
    if scale is not None:
        o = o * scale
    if act:
        o = jnp.maximum(o, 0.0)
    o_ref[...] = o.astype(o_ref.dtype)


def _mm_bias_body(a_ref, b_ref, bias_ref, o_ref, *, act):
    o = _dot(a_ref[...], b_ref[...]) + bias_ref[...]
    if act:
        o = jnp.maximum(o, 0.0)
    o_ref[...] = o.astype(o_ref.dtype)


def _mm(a, b, bias=None, act=False, out_dtype=_F32, block_m=512, scale=None):
    m, k = a.shape
    _, n = b.shape
    in_specs = [
        pl.BlockSpec((block_m, k), lambda i: (i, 0)),
        pl.BlockSpec((k, n), lambda i: (0, 0)),
    ]
    args = [a, b]
    if bias is not None:
        in_specs.append(pl.BlockSpec((1, n), lambda i: (0, 0)))
        args.append(bias.reshape(1, n))
        body = functools.partial(_mm_bias_body, act=act)
    else:
        body = functools.partial(_mm_body, act=act, scale=scale)
    return pl.pallas_call(
        body,
        grid=(m // block_m,),
        in_specs=in_specs,
        out_specs=pl.BlockSpec((block_m, n), lambda i: (i, 0)),
        out_shape=jax.ShapeDtypeStruct((m, n), out_dtype),
    )(*args)


def _hct_body(adj_ref, xw_ref, b3_ref, w1_ref, w2_ref, t1_ref, t2_ref):
    h1 = jnp.maximum(_dot(adj_ref[...], xw_ref[...]) + b3_ref[...], 0.0)
    t1_ref[...] = _dot(h1, w1_ref[...]).astype(t1_ref.dtype)
    t2_ref[...] = _dot(h1, w2_ref[...]).astype(t2_ref.dtype)


def _hct(adj, xw, b3, w1, w2, block_m=512):
    m, k = adj.shape
    kh, n = w1.shape
    return pl.pallas_call(
        _hct_body,
        grid=(m // block_m,),
        in_specs=[
            pl.BlockSpec((block_m, k), lambda i: (i, 0)),
            pl.BlockSpec((k, kh), lambda i: (0, 0)),
            pl.BlockSpec((1, kh), lambda i: (0, 0)),
            pl.BlockSpec((kh, n), lambda i: (0, 0)),
            pl.BlockSpec((kh, n), lambda i: (0, 0)),
        ],
        out_specs=[
            pl.BlockSpec((block_m, n), lambda i: (i, 0)),
            pl.BlockSpec((block_m, n), lambda i: (i, 0)),
        ],
        out_shape=[
            jax.ShapeDtypeStruct((m, n), _BF16),
            jax.ShapeDtypeStruct((m, n), _BF16),
        ],
    )(adj, xw, b3.reshape(1, kh), w1, w2)


def _mm2_body(a_ref, b1_ref, b2_ref, c1_ref, c2_ref, o1_ref, o2_ref):
    a = a_ref[...].astype(_BF16)
    o1 = _dot(a, b1_ref[...]) + c1_ref[...]
    o2 = _dot(a, b2_ref[...]) + c2_ref[...]
    o1_ref[...] = o1.astype(o1_ref.dtype)
    o2_ref[...] = o2.astype(o2_ref.dtype)


def _mm2(a, b1, b2, c1, c2, out_dtype=_F32, block_m=256):
    m, k = a.shape
    _, n = b1.shape
    return pl.pallas_call(
        _mm2_body,
        grid=(m // block_m,),
        in_specs=[
            pl.BlockSpec((block_m, k), lambda i: (i, 0)),
            pl.BlockSpec((k, n), lambda i: (0, 0)),
            pl.BlockSpec((k, n), lambda i: (0, 0)),
            pl.BlockSpec((1, n), lambda i: (0, 0)),
            pl.BlockSpec((1, n), lambda i: (0, 0)),
        ],
        out_specs=[
            pl.BlockSpec((block_m, n), lambda i: (i, 0)),
            pl.BlockSpec((block_m, n), lambda i: (i, 0)),
        ],
        out_shape=[
            jax.ShapeDtypeStruct((m, n), out_dtype),
            jax.ShapeDtypeStruct((m, n), out_dtype),
        ],
    )(a, b1, b2, c1.reshape(1, n), c2.reshape(1, n))


def _muz_body(g1a_ref, g1b_ref, g2a_ref, g2b_ref, eps_ref, mu_ref, lv_ref, z_ref):
    mu = _dot(g1a_ref[...], g1b_ref[...])
    lv = _dot(g2a_ref[...], g2b_ref[...])
    mu_ref[...] = mu
    lv_ref[...] = lv
    z = mu + eps_ref[...].astype(_F32) * jnp.exp(0.5 * lv)
    z_ref[...] = z.astype(z_ref.dtype)


def _muz(g1, g2, eps, block_m=512):
    n = g1.shape[0]
    row = pl.BlockSpec((block_m, n), lambda i: (i, 0))
    full = pl.BlockSpec((n, n), lambda i: (0, 0))
    return pl.pallas_call(
        _muz_body,
        grid=(n // block_m,),
        in_specs=[row, full, row, full, row],
        out_specs=[row, row, row],
        out_shape=[
            jax.ShapeDtypeStruct((n, n), _F32),
            jax.ShapeDtypeStruct((n, n), _F32),
            jax.ShapeDtypeStruct((n, n), _BF16),
        ],
    )(g1, g1, g2, g2, eps)


def _sym_nt_body(
    i_ref, j_ref, a1_ref, a2_ref, o_ref,
    blk0, blkt0, blk1, blkt1, sem10, sem20, sem11, sem21,
    *, act, block, nsteps,
):
    t = pl.program_id(0)

    def dmas(s, blk, blkt, s1, s2):
        si = i_ref[s]
        sj = j_ref[s]
        cp1 = pltpu.make_async_copy(
            blk, o_ref.at[pl.ds(si * block, block), pl.ds(sj * block, block)], s1
        )
        cp2 = pltpu.make_async_copy(
            blkt, o_ref.at[pl.ds(sj * block, block), pl.ds(si * block, block)], s2
        )
        return cp1, cp2, si != sj

    def wait_step(s, blk, blkt, s1, s2):
        cp1, cp2, offdiag = dmas(s, blk, blkt, s1, s2)
        cp1.wait()

        @pl.when(offdiag)
        def _():
            cp2.wait()

    def run(blk, blkt, s1, s2, o_blk, o_blkt):
        @pl.when(t >= 2)
        def _():
            wait_step(t - 2, blk, blkt, s1, s2)

        blk[...] = o_blk
        cp1, cp2, offdiag = dmas(t, blk, blkt, s1, s2)
        cp1.start()

        @pl.when(offdiag)
        def _():
            blkt[...] = o_blkt
            cp2.start()

    o = _dot(a1_ref[...], a2_ref[...], trans_b=True)
    if act:
        o = jnp.maximum(o, 0.0)
    oc = o.astype(blk0.dtype)
    oct = oc.T

    @pl.when(t % 2 == 0)
    def _even():
        run(blk0, blkt0, sem10, sem20, oc, oct)

    @pl.when(t % 2 == 1)
    def _odd():
        run(blk1, blkt1, sem11, sem21, oc, oct)

    @pl.when(t == nsteps - 1)
    def _drain():
        @pl.when(t % 2 == 0)
        def _():
            wait_step(t, blk0, blkt0, sem10, sem20)
            wait_step(t - 1, blk1, blkt1, sem11, sem21)

        @pl.when(t % 2 == 1)
        def _():
            wait_step(t, blk1, blkt1, sem11, sem21)
            wait_step(t - 1, blk0, blkt0, sem10, sem20)


def _sym_nt(a, act=False, out_dtype=_F32, block=512):
    m, k = a.shape
    nb = m // block
    pairs = [(i, j) for i in range(nb) for j in range(nb) if i <= j]
    i_arr = jnp.asarray(np.array([p[0] for p in pairs], dtype=np.int32))
    j_arr = jnp.asarray(np.array([p[1] for p in pairs], dtype=np.int32))
    grid_spec = pltpu.PrefetchScalarGridSpec(
        num_scalar_prefetch=2,
        grid=(len(pairs),),
        in_specs=[
            pl.BlockSpec((block, k), lambda t, i_arr, j_arr: (i_arr[t], 0)),
            pl.BlockSpec((block, k), lambda t, i_arr, j_arr: (j_arr[t], 0)),
        ],
        out_specs=pl.BlockSpec(memory_space=pl.ANY),
        scratch_shapes=[
            pltpu.VMEM((block, block), out_dtype),
            pltpu.VMEM((block, block), out_dtype),
            pltpu.VMEM((block, block), out_dtype),
            pltpu.VMEM((block, block), out_dtype),
            pltpu.SemaphoreType.DMA,
            pltpu.SemaphoreType.DMA,
            pltpu.SemaphoreType.DMA,
            pltpu.SemaphoreType.DMA,
        ],
    )
    return pl.pallas_call(
        functools.partial(_sym_nt_body, act=act, block=block, nsteps=len(pairs)),
        grid_spec=grid_spec,
        out_shape=jax.ShapeDtypeStruct((m, m), out_dtype),
    )(i_arr, j_arr, a, a)


def _zzy_body(
    i_ref, j_ref, z1_ref, z2_ref, y_ref,
    zz_vmem, blk0, blkt0, blk1, blkt1, sem10, sem20, sem11, sem21,
    *, block, npairs,
):
    t = pl.program_id(0)
    u = t % npairs
    i = i_ref[u]
    j = j_ref[u]

    @pl.when(t < npairs)
    def _build_zz():
        o = _dot(z1_ref[...], z2_ref[...], trans_b=True)
        oc = o.astype(_BF16)
        zz_vmem[pl.ds(i * block, block), pl.ds(j * block, block)] = oc

        @pl.when(i != j)
        def _():
            zz_vmem[pl.ds(j * block, block), pl.ds(i * block, block)] = oc.T

    @pl.when(t >= npairs)
    def _build_y():
        def dmas(s, blk, blkt, s1, s2):
            si = i_ref[s % npairs]
            sj = j_ref[s % npairs]
            cp1 = pltpu.make_async_copy(
                blk, y_ref.at[pl.ds(si * block, block), pl.ds(sj * block, block)], s1
            )
            cp2 = pltpu.make_async_copy(
                blkt, y_ref.at[pl.ds(sj * block, block), pl.ds(si * block, block)], s2
            )
            return cp1, cp2, si != sj

        def wait_step(s, blk, blkt, s1, s2):
            cp1, cp2, offdiag = dmas(s, blk, blkt, s1, s2)
            cp1.wait()

            @pl.when(offdiag)
            def _():
                cp2.wait()

        a = zz_vmem[pl.ds(i * block, block), :]
        b = zz_vmem[pl.ds(j * block, block), :]
        o = jnp.maximum(_dot(a, b, trans_b=True), 0.0)
        oct = o.T

        def run(blk, blkt, s1, s2):
            @pl.when(t - npairs >= 2)
            def _():
                wait_step(t - 2, blk, blkt, s1, s2)

            blk[...] = o
            cp1, cp2, offdiag = dmas(t, blk, blkt, s1, s2)
            cp1.start()

            @pl.when(offdiag)
            def _():
                blkt[...] = oct
                cp2.start()

        @pl.when(t % 2 == 0)
        def _even():
            run(blk0, blkt0, sem10, sem20)

        @pl.when(t % 2 == 1)
        def _odd():
            run(blk1, blkt1, sem11, sem21)

        @pl.when(t == 2 * npairs - 1)
        def _drain():
            @pl.when(t % 2 == 0)
            def _():
                wait_step(t, blk0, blkt0, sem10, sem20)
                wait_step(t - 1, blk1, blkt1, sem11, sem21)

            @pl.when(t % 2 == 1)
            def _():
                wait_step(t, blk1, blkt1, sem11, sem21)
                wait_step(t - 1, blk0, blkt0, sem10, sem20)


def _zzy(z, block=512):
    m, k = z.shape
    nb = m // block
    pairs = [(i, j) for i in range(nb) for j in range(nb) if i <= j]
    npairs = len(pairs)
    i_arr = jnp.asarray(np.array([p[0] for p in pairs], dtype=np.int32))
    j_arr = jnp.asarray(np.array([p[1] for p in pairs], dtype=np.int32))

    def zspec(sel):
        return pl.BlockSpec(
            (block, k),
            lambda t, i_arr, j_arr: (
                jnp.where(t < npairs, sel(i_arr, j_arr)[t % npairs], 0),
                0,
            ),
        )

    grid_spec = pltpu.PrefetchScalarGridSpec(
        num_scalar_prefetch=2,
        grid=(2 * npairs,),
        in_specs=[
            zspec(lambda ia, ja: ia),
            zspec(lambda ia, ja: ja),
        ],
        out_specs=pl.BlockSpec(memory_space=pl.ANY),
        scratch_shapes=[
            pltpu.VMEM((m, m), _BF16),
            pltpu.VMEM((block, block), _F32),
            pltpu.VMEM((block, block), _F32),
            pltpu.VMEM((block, block), _F32),
            pltpu.VMEM((block, block), _F32),
            pltpu.SemaphoreType.DMA,
            pltpu.SemaphoreType.DMA,
            pltpu.SemaphoreType.DMA,
            pltpu.SemaphoreType.DMA,
        ],
    )
    return pl.pallas_call(
        functools.partial(_zzy_body, block=block, npairs=npairs),
        grid_spec=grid_spec,
        out_shape=jax.ShapeDtypeStruct((m, m), _F32),
    )(i_arr, j_arr, z, z)


def kernel(x, adj, W1, b1, W2, b2, W3, b3):
    n = adj.shape[0]
    xw = _mm(x, W3, out_dtype=_BF16)
    t1, t2 = _hct(adj, xw, b3, W1, W2)
    g1, g2 = _mm2(adj, t1, t2, b1, b2, out_dtype=_BF16)
    eps = jax.random.uniform(jax.random.key(42), (n, n), dtype=_F32)
    mu, logvar, z = _muz(g1, g2, eps.astype(_BF16))
    y = _zzy(z)
    return (mu, logvar, y)


# speedup vs baseline: 1.0434x; 1.0434x over previous
import functools

import jax
import jax.numpy as jnp
import numpy as np
from jax import lax
from jax.experimental import pallas as pl
from jax.experimental.pallas import tpu as pltpu

_F32 = jnp.float32
_BF16 = jnp.bfloat16
_F8 = jnp.float8_e4m3fn


def _dot(a, b, trans_b=False):
    if a.dtype != _F8:
        a = a.astype(_BF16)
    if b.dtype != _F8:
        b = b.astype(_BF16)
    dims = (((1,), (1 if trans_b else 0,)), ((), ()))
    return lax.dot_general(a, b, dims, preferred_element_type=_F32)


def _mm_body(a_ref, b_ref, o_ref, *, act, scale):
    o = _dot(a_ref[...], b_ref[...])
    if scale is not None:
        o = o * scale
    if act:
        o = jnp.maximum(o, 0.0)
    o_ref[...] = o.astype(o_ref.dtype)


def _mm_bias_body(a_ref, b_ref, bias_ref, o_ref, *, act):
    o = _dot(a_ref[...], b_ref[...]) + bias_ref[...]
    if act:
        o = jnp.maximum(o, 0.0)
    o_ref[...] = o.astype(o_ref.dtype)


def _mm(a, b, bias=None, act=False, out_dtype=_F32, block_m=512, scale=None):
    m, k = a.shape
    _, n = b.shape
    in_specs = [
        pl.BlockSpec((block_m, k), lambda i: (i, 0)),
        pl.BlockSpec((k, n), lambda i: (0, 0)),
    ]
    args = [a, b]
    if bias is not None:
        in_specs.append(pl.BlockSpec((1, n), lambda i: (0, 0)))
        args.append(bias.reshape(1, n))
        body = functools.partial(_mm_bias_body, act=act)
    else:
        body = functools.partial(_mm_body, act=act, scale=scale)
    return pl.pallas_call(
        body,
        grid=(m // block_m,),
        in_specs=in_specs,
        out_specs=pl.BlockSpec((block_m, n), lambda i: (i, 0)),
        out_shape=jax.ShapeDtypeStruct((m, n), out_dtype),
    )(*args)


def _hct_body(adj_ref, xw_ref, b3_ref, w1_ref, w2_ref, t1_ref, t2_ref):
    h1 = jnp.maximum(_dot(adj_ref[...], xw_ref[...]) + b3_ref[...], 0.0)
    t1_ref[...] = _dot(h1, w1_ref[...]).astype(t1_ref.dtype)
    t2_ref[...] = _dot(h1, w2_ref[...]).astype(t2_ref.dtype)


def _hct(adj, xw, b3, w1, w2, block_m=512):
    m, k = adj.shape
    kh, n = w1.shape
    return pl.pallas_call(
        _hct_body,
        grid=(m // block_m,),
        in_specs=[
            pl.BlockSpec((block_m, k), lambda i: (i, 0)),
            pl.BlockSpec((k, kh), lambda i: (0, 0)),
            pl.BlockSpec((1, kh), lambda i: (0, 0)),
            pl.BlockSpec((kh, n), lambda i: (0, 0)),
            pl.BlockSpec((kh, n), lambda i: (0, 0)),
        ],
        out_specs=[
            pl.BlockSpec((block_m, n), lambda i: (i, 0)),
            pl.BlockSpec((block_m, n), lambda i: (i, 0)),
        ],
        out_shape=[
            jax.ShapeDtypeStruct((m, n), _BF16),
            jax.ShapeDtypeStruct((m, n), _BF16),
        ],
    )(adj, xw, b3.reshape(1, kh), w1, w2)


def _gmuz_body(
    adj_ref, t1_ref, t2_ref, b1_ref, b2_ref, eps_ref,
    mu_ref, lv_ref, z_ref, g1_s, g2_s,
    *, gblock, mblock, gsteps,
):
    t = pl.program_id(0)

    @pl.when(t < gsteps)
    def _build_g():
        a = adj_ref[...].astype(_BF16)
        g1 = _dot(a, t1_ref[...]) + b1_ref[...]
        g2 = _dot(a, t2_ref[...]) + b2_ref[...]
        g1_s[pl.ds(t * gblock, gblock), :] = g1.astype(_BF16)
        g2_s[pl.ds(t * gblock, gblock), :] = g2.astype(_BF16)

    @pl.when(t >= gsteps)
    def _muz():
        u = t - gsteps
        g1r = g1_s[pl.ds(u * mblock, mblock), :]
        g2r = g2_s[pl.ds(u * mblock, mblock), :]
        mu = _dot(g1r, g1_s[...])
        lv = _dot(g2r, g2_s[...])
        mu_ref[...] = mu
        lv_ref[...] = lv
        z = mu + eps_ref[...].astype(_F32) * jnp.exp(0.5 * lv)
        z_ref[...] = z.astype(_BF16)


def _gmuz(adj, t1, t2, b1, b2, eps, gblock=512, mblock=256):
    m, k = adj.shape
    n = t1.shape[1]
    gsteps = m // gblock
    msteps = m // mblock
    grid = (gsteps + msteps,)
    last_g = gsteps - 1
    last_m = msteps - 1

    def clampg(t):
        return jnp.where(t < gsteps, jnp.minimum(t, last_g), last_g)

    def clampm(t):
        return jnp.clip(t - gsteps, 0, last_m)

    row_out = pl.BlockSpec((mblock, n), lambda t: (clampm(t), 0))
    return pl.pallas_call(
        functools.partial(
            _gmuz_body, gblock=gblock, mblock=mblock, gsteps=gsteps
        ),
        grid=grid,
        in_specs=[
            pl.BlockSpec((gblock, k), lambda t: (clampg(t), 0)),
            pl.BlockSpec((k, n), lambda t: (0, 0)),
            pl.BlockSpec((k, n), lambda t: (0, 0)),
            pl.BlockSpec((1, n), lambda t: (0, 0)),
            pl.BlockSpec((1, n), lambda t: (0, 0)),
            pl.BlockSpec((mblock, n), lambda t: (clampm(t), 0)),
        ],
        out_specs=[row_out, row_out, row_out],
        out_shape=[
            jax.ShapeDtypeStruct((m, n), _F32),
            jax.ShapeDtypeStruct((m, n), _F32),
            jax.ShapeDtypeStruct((m, n), _BF16),
        ],
        scratch_shapes=[
            pltpu.VMEM((m, n), _BF16),
            pltpu.VMEM((m, n), _BF16),
        ],
    )(adj, t1, t2, b1.reshape(1, n), b2.reshape(1, n), eps)


def _muz_body(g1a_ref, g1b_ref, g2a_ref, g2b_ref, eps_ref, mu_ref, lv_ref, z_ref):
    mu = _dot(g1a_ref[...], g1b_ref[...])
    lv = _dot(g2a_ref[...], g2b_ref[...])
    mu_ref[...] = mu
    lv_ref[...] = lv
    z = mu + eps_ref[...].astype(_F32) * jnp.exp(0.5 * lv)
    z_ref[...] = z.astype(z_ref.dtype)


def _muz(g1, g2, eps, block_m=512):
    n = g1.shape[0]
    row = pl.BlockSpec((block_m, n), lambda i: (i, 0))
    full = pl.BlockSpec((n, n), lambda i: (0, 0))
    return pl.pallas_call(
        _muz_body,
        grid=(n // block_m,),
        in_specs=[row, full, row, full, row],
        out_specs=[row, row, row],
        out_shape=[
            jax.ShapeDtypeStruct((n, n), _F32),
            jax.ShapeDtypeStruct((n, n), _F32),
            jax.ShapeDtypeStruct((n, n), _BF16),
        ],
    )(g1, g1, g2, g2, eps)


def _sym_nt_body(
    i_ref, j_ref, a1_ref, a2_ref, o_ref,
    blk0, blkt0, blk1, blkt1, sem10, sem20, sem11, sem21,
    *, act, block, nsteps,
):
    t = pl.program_id(0)

    def dmas(s, blk, blkt, s1, s2):
        si = i_ref[s]
        sj = j_ref[s]
        cp1 = pltpu.make_async_copy(
            blk, o_ref.at[pl.ds(si * block, block), pl.ds(sj * block, block)], s1
        )
        cp2 = pltpu.make_async_copy(
            blkt, o_ref.at[pl.ds(sj * block, block), pl.ds(si * block, block)], s2
        )
        return cp1, cp2, si != sj

    def wait_step(s, blk, blkt, s1, s2):
        cp1, cp2, offdiag = dmas(s, blk, blkt, s1, s2)
        cp1.wait()

        @pl.when(offdiag)
        def _():
            cp2.wait()

    def run(blk, blkt, s1, s2, o_blk, o_blkt):
        @pl.when(t >= 2)
        def _():
            wait_step(t - 2, blk, blkt, s1, s2)

        blk[...] = o_blk
        cp1, cp2, offdiag = dmas(t, blk, blkt, s1, s2)
        cp1.start()

        @pl.when(offdiag)
        def _():
            blkt[...] = o_blkt
            cp2.start()

    o = _dot(a1_ref[...], a2_ref[...], trans_b=True)
    if act:
        o = jnp.maximum(o, 0.0)
    oc = o.astype(blk0.dtype)
    oct = oc.T

    @pl.when(t % 2 == 0)
    def _even():
        run(blk0, blkt0, sem10, sem20, oc, oct)

    @pl.when(t % 2 == 1)
    def _odd():
        run(blk1, blkt1, sem11, sem21, oc, oct)

    @pl.when(t == nsteps - 1)
    def _drain():
        @pl.when(t % 2 == 0)
        def _():
            wait_step(t, blk0, blkt0, sem10, sem20)
            wait_step(t - 1, blk1, blkt1, sem11, sem21)

        @pl.when(t % 2 == 1)
        def _():
            wait_step(t, blk1, blkt1, sem11, sem21)
            wait_step(t - 1, blk0, blkt0, sem10, sem20)


def _sym_nt(a, act=False, out_dtype=_F32, block=512):
    m, k = a.shape
    nb = m // block
    pairs = [(i, j) for i in range(nb) for j in range(nb) if i <= j]
    i_arr = jnp.asarray(np.array([p[0] for p in pairs], dtype=np.int32))
    j_arr = jnp.asarray(np.array([p[1] for p in pairs], dtype=np.int32))
    grid_spec = pltpu.PrefetchScalarGridSpec(
        num_scalar_prefetch=2,
        grid=(len(pairs),),
        in_specs=[
            pl.BlockSpec((block, k), lambda t, i_arr, j_arr: (i_arr[t], 0)),
            pl.BlockSpec((block, k), lambda t, i_arr, j_arr: (j_arr[t], 0)),
        ],
        out_specs=pl.BlockSpec(memory_space=pl.ANY),
        scratch_shapes=[
            pltpu.VMEM((block, block), out_dtype),
            pltpu.VMEM((block, block), out_dtype),
            pltpu.VMEM((block, block), out_dtype),
            pltpu.VMEM((block, block), out_dtype),
            pltpu.SemaphoreType.DMA,
            pltpu.SemaphoreType.DMA,
            pltpu.SemaphoreType.DMA,
            pltpu.SemaphoreType.DMA,
        ],
    )
    return pl.pallas_call(
        functools.partial(_sym_nt_body, act=act, block=block, nsteps=len(pairs)),
        grid_spec=grid_spec,
        out_shape=jax.ShapeDtypeStruct((m, m), out_dtype),
    )(i_arr, j_arr, a, a)


def _zzy_body(
    i_ref, j_ref, z1_ref, z2_ref, y_ref,
    zz_vmem, blk0, blkt0, blk1, blkt1, sem10, sem20, sem11, sem21,
    *, block, npairs,
):
    t = pl.program_id(0)
    u = t % npairs
    i = i_ref[u]
    j = j_ref[u]

    @pl.when(t < npairs)
    def _build_zz():
        o = _dot(z1_ref[...], z2_ref[...], trans_b=True)
        oc = o.astype(_BF16)
        zz_vmem[pl.ds(i * block, block), pl.ds(j * block, block)] = oc

        @pl.when(i != j)
        def _():
            zz_vmem[pl.ds(j * block, block), pl.ds(i * block, block)] = oc.T

    @pl.when(t >= npairs)
    def _build_y():
        def dmas(s, blk, blkt, s1, s2):
            si = i_ref[s % npairs]
            sj = j_ref[s % npairs]
            cp1 = pltpu.make_async_copy(
                blk, y_ref.at[pl.ds(si * block, block), pl.ds(sj * block, block)], s1
            )
            cp2 = pltpu.make_async_copy(
                blkt, y_ref.at[pl.ds(sj * block, block), pl.ds(si * block, block)], s2
            )
            return cp1, cp2, si != sj

        def wait_step(s, blk, blkt, s1, s2):
            cp1, cp2, offdiag = dmas(s, blk, blkt, s1, s2)
            cp1.wait()

            @pl.when(offdiag)
            def _():
                cp2.wait()

        a = zz_vmem[pl.ds(i * block, block), :]
        b = zz_vmem[pl.ds(j * block, block), :]
        o = jnp.maximum(_dot(a, b, trans_b=True), 0.0)
        oct = o.T

        def run(blk, blkt, s1, s2):
            @pl.when(t - npairs >= 2)
            def _():
                wait_step(t - 2, blk, blkt, s1, s2)

            blk[...] = o
            cp1, cp2, offdiag = dmas(t, blk, blkt, s1, s2)
            cp1.start()

            @pl.when(offdiag)
            def _():
                blkt[...] = oct
                cp2.start()

        @pl.when(t % 2 == 0)
        def _even():
            run(blk0, blkt0, sem10, sem20)

        @pl.when(t % 2 == 1)
        def _odd():
            run(blk1, blkt1, sem11, sem21)

        @pl.when(t == 2 * npairs - 1)
        def _drain():
            @pl.when(t % 2 == 0)
            def _():
                wait_step(t, blk0, blkt0, sem10, sem20)
                wait_step(t - 1, blk1, blkt1, sem11, sem21)

            @pl.when(t % 2 == 1)
            def _():
                wait_step(t, blk1, blkt1, sem11, sem21)
                wait_step(t - 1, blk0, blkt0, sem10, sem20)


def _zzy(z, block=512):
    m, k = z.shape
    nb = m // block
    pairs = [(i, j) for i in range(nb) for j in range(nb) if i <= j]
    npairs = len(pairs)
    i_arr = jnp.asarray(np.array([p[0] for p in pairs], dtype=np.int32))
    j_arr = jnp.asarray(np.array([p[1] for p in pairs], dtype=np.int32))

    def zspec(sel):
        return pl.BlockSpec(
            (block, k),
            lambda t, i_arr, j_arr: (
                jnp.where(t < npairs, sel(i_arr, j_arr)[t % npairs], 0),
                0,
            ),
        )

    grid_spec = pltpu.PrefetchScalarGridSpec(
        num_scalar_prefetch=2,
        grid=(2 * npairs,),
        in_specs=[
            zspec(lambda ia, ja: ia),
            zspec(lambda ia, ja: ja),
        ],
        out_specs=pl.BlockSpec(memory_space=pl.ANY),
        scratch_shapes=[
            pltpu.VMEM((m, m), _BF16),
            pltpu.VMEM((block, block), _F32),
            pltpu.VMEM((block, block), _F32),
            pltpu.VMEM((block, block), _F32),
            pltpu.VMEM((block, block), _F32),
            pltpu.SemaphoreType.DMA,
            pltpu.SemaphoreType.DMA,
            pltpu.SemaphoreType.DMA,
            pltpu.SemaphoreType.DMA,
        ],
    )
    return pl.pallas_call(
        functools.partial(_zzy_body, block=block, npairs=npairs),
        grid_spec=grid_spec,
        out_shape=jax.ShapeDtypeStruct((m, m), _F32),
    )(i_arr, j_arr, z, z)


def kernel(x, adj, W1, b1, W2, b2, W3, b3):
    n = adj.shape[0]
    xw = _mm(x, W3, out_dtype=_BF16)
    t1, t2 = _hct(adj, xw, b3, W1, W2)
    eps = jax.random.uniform(jax.random.key(42), (n, n), dtype=_F32)
    mu, logvar, z = _gmuz(adj, t1, t2, b1, b2, eps.astype(_BF16))
    y = _zzy(z)
    return (mu, logvar, y)
